# Initial kernel scaffold; baseline (speedup 1.0000x reference)
import jax
import jax.numpy as jnp
from jax.experimental import pallas as pl


def kernel(f_atoms, f_bonds, a2b, b2a, b2revb, seg, W_i, W_h, W_o, b_o):
    raise NotImplementedError("write your pallas kernel here")


# speedup vs baseline: 1.3126x; 1.3126x over previous
import functools

import jax
import jax.numpy as jnp
from jax import lax
from jax.experimental import pallas as pl
from jax.experimental.pallas import tpu as pltpu
from jax.experimental.pallas import tpu_sc as plsc

H = 128
NW = 32
LANES = 16


def _wid():
    return lax.axis_index("s") * 2 + lax.axis_index("c")


def _mesh():
    return plsc.VectorSubcoreMesh(core_axis_name="c", subcore_axis_name="s")


def _mm0_body(fb, wi, wh, inp_o, hm_o):
    inp = jnp.dot(fb[...], wi[...], preferred_element_type=jnp.float32)
    inp_o[...] = inp
    hm_o[...] = jnp.dot(jnp.maximum(inp, 0.0), wh[...],
                        preferred_element_type=jnp.float32)


def _tc_mm0(f_bonds, W_i, W_h):
    nb = f_bonds.shape[0]
    B = 2000
    return pl.pallas_call(
        _mm0_body,
        grid=(nb // B,),
        in_specs=[
            pl.BlockSpec((B, f_bonds.shape[1]), lambda i: (i, 0)),
            pl.BlockSpec(W_i.shape, lambda i: (0, 0)),
            pl.BlockSpec(W_h.shape, lambda i: (0, 0)),
        ],
        out_specs=[
            pl.BlockSpec((B, H), lambda i: (i, 0)),
            pl.BlockSpec((B, H), lambda i: (i, 0)),
        ],
        out_shape=[
            jax.ShapeDtypeStruct((nb, H), jnp.float32),
            jax.ShapeDtypeStruct((nb, H), jnp.float32),
        ],
    )(f_bonds, W_i, W_h)


def _mm1_body(inp, g, wh, hm_o):
    m = jnp.maximum(inp[...] + g[...], 0.0)
    hm_o[...] = jnp.dot(m, wh[...], preferred_element_type=jnp.float32)


def _tc_mm1(inp, g, W_h):
    nb = inp.shape[0]
    B = 2000
    return pl.pallas_call(
        _mm1_body,
        grid=(nb // B,),
        in_specs=[
            pl.BlockSpec((B, H), lambda i: (i, 0)),
            pl.BlockSpec((B, H), lambda i: (i, 0)),
            pl.BlockSpec(W_h.shape, lambda i: (0, 0)),
        ],
        out_specs=pl.BlockSpec((B, H), lambda i: (i, 0)),
        out_shape=jax.ShapeDtypeStruct((nb, H), jnp.float32),
    )(inp, g, W_h)


def _sc_gathersum(hm, a2b_flat):
    nrows = a2b_flat.shape[0]
    n_at = nrows // 16
    CA = 8
    RPC = CA * 16
    n_chunks = n_at // CA
    base_q, rem = divmod(n_chunks, NW)

    @functools.partial(
        pl.kernel,
        out_type=jax.ShapeDtypeStruct((n_at, H), jnp.float32),
        mesh=_mesh(),
        scratch_types=[
            pltpu.VMEM((RPC,), jnp.int32),
            pltpu.VMEM((RPC, H), jnp.float32),
            pltpu.VMEM((CA, H), jnp.float32),
            pltpu.SemaphoreType.DMA,
        ],
    )
    def k(hm_hbm, idx_hbm, out_hbm, idx_v, rows_v, out_v, sem):
        w = _wid()
        nc = base_q + jnp.where(w < rem, 1, 0)
        c0 = w * base_q + jnp.minimum(w, rem)

        def body(ci, _):
            c = c0 + ci
            pltpu.sync_copy(idx_hbm.at[pl.ds(c * RPC, RPC)], idx_v)
            pltpu.async_copy(hm_hbm.at[idx_v], rows_v, sem).wait()
            for a in range(CA):
                for s in range(H // LANES):
                    sl = pl.ds(s * LANES, LANES)
                    acc = rows_v[a * 16, sl]
                    for j in range(1, 16):
                        acc = acc + rows_v[a * 16 + j, sl]
                    out_v[a, sl] = acc
            pltpu.sync_copy(out_v, out_hbm.at[pl.ds(c * CA, CA)])
            return 0

        lax.fori_loop(0, nc, body, 0)

    return k(hm, a2b_flat)


def _sc_combine(ahm, hm, b2a, b2revb, inp=None):
    nb = hm.shape[0]
    per_w = nb // NW
    R = 128
    n_full = per_w // R
    with_inp = inp is not None

    scratch = [
        pltpu.VMEM((R,), jnp.int32),
        pltpu.VMEM((R,), jnp.int32),
        pltpu.VMEM((R, H), jnp.float32),
        pltpu.VMEM((R, H), jnp.float32),
        pltpu.SemaphoreType.DMA,
        pltpu.SemaphoreType.DMA,
    ]
    if with_inp:
        scratch.append(pltpu.VMEM((R, H), jnp.float32))

    def body(ahm_hbm, hm_hbm, b2a_hbm, b2revb_hbm, *rest):
        if with_inp:
            inp_hbm, out_hbm, ia_v, ib_v, ra_v, rb_v, sa, sb, ri_v = rest
        else:
            out_hbm, ia_v, ib_v, ra_v, rb_v, sa, sb = rest
        w = _wid()
        b0 = w * per_w

        def chunk(ci, _):
            start = b0 + jnp.where(ci < n_full, ci * R, per_w - R)
            pltpu.sync_copy(b2a_hbm.at[pl.ds(start, R)], ia_v)
            pltpu.sync_copy(b2revb_hbm.at[pl.ds(start, R)], ib_v)
            ca = pltpu.async_copy(ahm_hbm.at[ia_v], ra_v, sa)
            cb = pltpu.async_copy(hm_hbm.at[ib_v], rb_v, sb)
            if with_inp:
                pltpu.sync_copy(inp_hbm.at[pl.ds(start, R)], ri_v)
            ca.wait()
            cb.wait()

            def row(r, _):
                for s in range(H // LANES):
                    sl = pl.ds(s * LANES, LANES)
                    x = ra_v[r, sl] - rb_v[r, sl]
                    if with_inp:
                        x = jnp.maximum(x + ri_v[r, sl], 0.0)
                    ra_v[r, sl] = x
                return 0

            lax.fori_loop(0, R, row, 0)
            pltpu.sync_copy(ra_v, out_hbm.at[pl.ds(start, R)])
            return 0

        lax.fori_loop(0, n_full + 1, chunk, 0)

    kern = functools.partial(
        pl.kernel,
        out_type=jax.ShapeDtypeStruct((nb, H), jnp.float32),
        mesh=_mesh(),
        scratch_types=scratch,
    )(body)
    if with_inp:
        return kern(ahm, hm, b2a, b2revb, inp)
    return kern(ahm, hm, b2a, b2revb)


def _readout_body(fa, amr, segr, woa, wom, bor, out_ref, acc, cnt):
    i = pl.program_id(0)
    npg = pl.num_programs(0)

    @pl.when(i == 0)
    def _():
        acc[...] = jnp.zeros_like(acc)
        cnt[...] = jnp.zeros_like(cnt)

    ah = jnp.dot(fa[...], woa[...], preferred_element_type=jnp.float32)
    ah = ah + jnp.dot(amr[...], wom[...], preferred_element_type=jnp.float32)
    ah = jnp.maximum(ah + bor[...], 0.0)

    s = segr[0]
    A = s.shape[1]
    n_mols = acc.shape[0]
    MC = 500
    for h in range(n_mols // MC):
        iota = lax.broadcasted_iota(jnp.int32, (MC, A), 0) + h * MC
        ohT = (jnp.broadcast_to(s, (MC, A)) == iota).astype(jnp.float32)
        acc[pl.ds(h * MC, MC), :] += jnp.dot(
            ohT, ah, preferred_element_type=jnp.float32)
        cnt[pl.ds(h * MC, MC), :] += jnp.sum(ohT, axis=1, keepdims=True)

    @pl.when(i == npg - 1)
    def _():
        out_ref[...] = acc[...] / jnp.maximum(cnt[...], 1.0)


def _tc_readout(f_atoms, am, seg, W_o, b_o, n_mols=2000):
    na = f_atoms.shape[0]
    A = 1000
    seg3 = seg.reshape(na // A, 1, A)
    woa = W_o[:H]
    wom = W_o[H:]
    bor = b_o.reshape(1, H)
    return pl.pallas_call(
        _readout_body,
        grid=(na // A,),
        in_specs=[
            pl.BlockSpec((A, H), lambda i: (i, 0)),
            pl.BlockSpec((A, H), lambda i: (i, 0)),
            pl.BlockSpec((1, 1, A), lambda i: (i, 0, 0)),
            pl.BlockSpec((H, H), lambda i: (0, 0)),
            pl.BlockSpec((H, H), lambda i: (0, 0)),
            pl.BlockSpec((1, H), lambda i: (0, 0)),
        ],
        out_specs=pl.BlockSpec((n_mols, H), lambda i: (0, 0)),
        out_shape=jax.ShapeDtypeStruct((n_mols, H), jnp.float32),
        scratch_shapes=[
            pltpu.VMEM((n_mols, H), jnp.float32),
            pltpu.VMEM((n_mols, 1), jnp.float32),
        ],
        compiler_params=pltpu.CompilerParams(
            dimension_semantics=("arbitrary",)),
    )(f_atoms, am, seg3, woa, wom, bor)


def kernel(f_atoms, f_bonds, a2b, b2a, b2revb, seg, W_i, W_h, W_o, b_o):
    a2b_flat = a2b.reshape(-1)

    inp, hm = _tc_mm0(f_bonds, W_i, W_h)
    ahm0 = _sc_gathersum(hm, a2b_flat)
    g0 = _sc_combine(ahm0, hm, b2a, b2revb)
    hm1 = _tc_mm1(inp, g0, W_h)
    ahm1 = _sc_gathersum(hm1, a2b_flat)
    msg2 = _sc_combine(ahm1, hm1, b2a, b2revb, inp=inp)
    am2 = _sc_gathersum(msg2, a2b_flat)
    return _tc_readout(f_atoms, am2, seg, W_o, b_o)


# speedup vs baseline: 2.1043x; 1.6032x over previous
import functools

import jax
import jax.numpy as jnp
from jax import lax
from jax.experimental import pallas as pl
from jax.experimental.pallas import tpu as pltpu
from jax.experimental.pallas import tpu_sc as plsc

H = 128
NW = 32
LANES = 16


def _wid():
    return lax.axis_index("s") * 2 + lax.axis_index("c")


def _mesh():
    return plsc.VectorSubcoreMesh(core_axis_name="c", subcore_axis_name="s")


def _mm0_body(fb, wi, wh, inp_o, hm_o):
    inp = jnp.dot(fb[...], wi[...], preferred_element_type=jnp.float32)
    inp_o[...] = inp
    hm_o[...] = jnp.dot(jnp.maximum(inp, 0.0), wh[...],
                        preferred_element_type=jnp.float32)


def _tc_mm0(f_bonds, W_i, W_h):
    nb = f_bonds.shape[0]
    B = 4000
    return pl.pallas_call(
        _mm0_body,
        grid=(nb // B,),
        in_specs=[
            pl.BlockSpec((B, f_bonds.shape[1]), lambda i: (i, 0)),
            pl.BlockSpec(W_i.shape, lambda i: (0, 0)),
            pl.BlockSpec(W_h.shape, lambda i: (0, 0)),
        ],
        out_specs=[
            pl.BlockSpec((B, H), lambda i: (i, 0)),
            pl.BlockSpec((B, H), lambda i: (i, 0)),
        ],
        out_shape=[
            jax.ShapeDtypeStruct((nb, H), jnp.float32),
            jax.ShapeDtypeStruct((nb, H), jnp.float32),
        ],
    )(f_bonds, W_i, W_h)


def _mm1_body(inp, g, wh, hm_o):
    m = jnp.maximum(inp[...] + g[...], 0.0)
    hm_o[...] = jnp.dot(m, wh[...], preferred_element_type=jnp.float32)


def _tc_mm1(inp, g, W_h):
    nb = inp.shape[0]
    B = 4000
    return pl.pallas_call(
        _mm1_body,
        grid=(nb // B,),
        in_specs=[
            pl.BlockSpec((B, H), lambda i: (i, 0)),
            pl.BlockSpec((B, H), lambda i: (i, 0)),
            pl.BlockSpec(W_h.shape, lambda i: (0, 0)),
        ],
        out_specs=pl.BlockSpec((B, H), lambda i: (i, 0)),
        out_shape=jax.ShapeDtypeStruct((nb, H), jnp.float32),
    )(inp, g, W_h)


def _sc_gathersum(hm, a2b_flat):
    nrows = a2b_flat.shape[0]
    n_at = nrows // 16
    CA = 8
    RPC = CA * 16
    n_chunks = n_at // CA
    NC = -(-n_chunks // NW)
    if NC % 2:
        NC += 1
    SLAB = NC * RPC

    @functools.partial(
        pl.kernel,
        out_type=jax.ShapeDtypeStruct((n_at, H), jnp.float32),
        mesh=_mesh(),
        scratch_types=[
            pltpu.VMEM((SLAB,), jnp.int32),
            pltpu.VMEM((RPC, H), jnp.float32),
            pltpu.VMEM((RPC, H), jnp.float32),
            pltpu.VMEM((CA, H), jnp.float32),
            pltpu.VMEM((CA, H), jnp.float32),
            pltpu.SemaphoreType.DMA,
            pltpu.SemaphoreType.DMA,
            pltpu.SemaphoreType.DMA,
            pltpu.SemaphoreType.DMA,
        ],
    )
    def k(hm_hbm, idx_hbm, out_hbm, idx_s, rows0, rows1, out0, out1,
          sr0, sr1, so0, so1):
        rows = (rows0, rows1)
        outs = (out0, out1)
        srs = (sr0, sr1)
        sos = (so0, so1)
        w = _wid()
        c0 = (w * (n_chunks - NC)) // (NW - 1)

        pltpu.sync_copy(idx_hbm.at[pl.ds(c0 * RPC, SLAB)], idx_s)
        for p in range(2):
            pltpu.async_copy(
                hm_hbm.at[idx_s.at[pl.ds(p * RPC, RPC)]], rows[p], srs[p])

        def body(c2, _):
            for p in range(2):
                ci = 2 * c2 + p
                pltpu.make_async_copy(
                    hm_hbm.at[idx_s.at[pl.ds(0, RPC)]], rows[p],
                    srs[p]).wait()

                @pl.when(c2 > 0)
                def _():
                    pltpu.make_async_copy(
                        outs[p], out_hbm.at[pl.ds(0, CA)], sos[p]).wait()

                for a in range(CA):
                    for s in range(H // LANES):
                        sl = pl.ds(s * LANES, LANES)
                        acc = rows[p][a * 16, sl]
                        for j in range(1, 16):
                            acc = acc + rows[p][a * 16 + j, sl]
                        outs[p][a, sl] = acc
                pltpu.async_copy(
                    outs[p], out_hbm.at[pl.ds((c0 + ci) * CA, CA)], sos[p])

                @pl.when(ci + 2 < NC)
                def _():
                    pltpu.async_copy(
                        hm_hbm.at[idx_s.at[pl.ds((ci + 2) * RPC, RPC)]],
                        rows[p], srs[p])
            return 0

        lax.fori_loop(0, NC // 2, body, 0)
        for p in range(2):
            pltpu.make_async_copy(
                outs[p], out_hbm.at[pl.ds(0, CA)], sos[p]).wait()

    return k(hm, a2b_flat)


def _sc_combine(ahm, hm, b2a, b2revb, inp=None):
    nb = hm.shape[0]
    per_w = nb // NW
    R = 64
    n_full = per_w // R
    NC = n_full + 2
    with_inp = inp is not None

    scratch = [
        pltpu.VMEM((per_w,), jnp.int32),
        pltpu.VMEM((per_w,), jnp.int32),
        pltpu.VMEM((R, H), jnp.float32),
        pltpu.VMEM((R, H), jnp.float32),
        pltpu.VMEM((R, H), jnp.float32),
        pltpu.VMEM((R, H), jnp.float32),
        pltpu.VMEM((R, H), jnp.float32),
        pltpu.VMEM((R, H), jnp.float32),
        pltpu.SemaphoreType.DMA,
        pltpu.SemaphoreType.DMA,
        pltpu.SemaphoreType.DMA,
        pltpu.SemaphoreType.DMA,
    ]
    if with_inp:
        scratch += [
            pltpu.VMEM((R, H), jnp.float32),
            pltpu.VMEM((R, H), jnp.float32),
            pltpu.SemaphoreType.DMA,
            pltpu.SemaphoreType.DMA,
        ]

    def body(ahm_hbm, hm_hbm, b2a_hbm, b2revb_hbm, *rest):
        if with_inp:
            (inp_hbm, out_hbm, ia_s, ib_s, ra0, ra1, rb0, rb1, ou0, ou1,
             sr0, sr1, so0, so1, ri0, ri1, si0, si1) = rest
            ris = (ri0, ri1)
            sis = (si0, si1)
        else:
            (out_hbm, ia_s, ib_s, ra0, ra1, rb0, rb1, ou0, ou1,
             sr0, sr1, so0, so1) = rest
        ras = (ra0, ra1)
        rbs = (rb0, rb1)
        ous = (ou0, ou1)
        srs = (sr0, sr1)
        sos = (so0, so1)
        w = _wid()
        b0 = w * per_w

        def cstart(ci):
            return jnp.where(ci < n_full, ci * R, per_w - (NC - ci) * R)

        def fetch(ci, p):
            st = cstart(ci)
            pltpu.async_copy(ahm_hbm.at[ib_s.at[pl.ds(st, R)]],
                             ras[p], srs[p])
            pltpu.async_copy(hm_hbm.at[ia_s.at[pl.ds(st, R)]],
                             rbs[p], srs[p])
            if with_inp:
                pltpu.async_copy(inp_hbm.at[pl.ds(b0 + st, R)],
                                 ris[p], sis[p])

        pltpu.sync_copy(b2a_hbm.at[pl.ds(b0, per_w)], ib_s)
        pltpu.sync_copy(b2revb_hbm.at[pl.ds(b0, per_w)], ia_s)
        for p in range(2):
            fetch(p, p)

        def chunk(c2, _):
            for p in range(2):
                ci = 2 * c2 + p
                st = cstart(ci)
                for _ in range(2):
                    pltpu.make_async_copy(
                        hm_hbm.at[ia_s.at[pl.ds(0, R)]], rbs[p],
                        srs[p]).wait()
                if with_inp:
                    pltpu.make_async_copy(
                        inp_hbm.at[pl.ds(0, R)], ris[p], sis[p]).wait()

                @pl.when(c2 > 0)
                def _():
                    pltpu.make_async_copy(
                        ous[p], out_hbm.at[pl.ds(0, R)], sos[p]).wait()

                def row(r, _):
                    for s in range(H // LANES):
                        sl = pl.ds(s * LANES, LANES)
                        x = ras[p][r, sl] - rbs[p][r, sl]
                        if with_inp:
                            x = jnp.maximum(x + ris[p][r, sl], 0.0)
                        ous[p][r, sl] = x
                    return 0

                lax.fori_loop(0, R, row, 0)
                pltpu.async_copy(
                    ous[p], out_hbm.at[pl.ds(b0 + st, R)], sos[p])

                @pl.when(ci + 2 < NC)
                def _():
                    fetch(ci + 2, p)
            return 0

        lax.fori_loop(0, NC // 2, chunk, 0)
        for p in range(2):
            pltpu.make_async_copy(
                ous[p], out_hbm.at[pl.ds(0, R)], sos[p]).wait()

    kern = functools.partial(
        pl.kernel,
        out_type=jax.ShapeDtypeStruct((nb, H), jnp.float32),
        mesh=_mesh(),
        scratch_types=scratch,
    )(body)
    if with_inp:
        return kern(ahm, hm, b2a, b2revb, inp)
    return kern(ahm, hm, b2a, b2revb)


def _readout_body(fa, amr, segr, woa, wom, bor, out_ref, acc, cnt):
    i = pl.program_id(0)
    npg = pl.num_programs(0)

    @pl.when(i == 0)
    def _():
        acc[...] = jnp.zeros_like(acc)
        cnt[...] = jnp.zeros_like(cnt)

    ah = jnp.dot(fa[...], woa[...], preferred_element_type=jnp.float32)
    ah = ah + jnp.dot(amr[...], wom[...], preferred_element_type=jnp.float32)
    ah = jnp.maximum(ah + bor[...], 0.0)

    s = segr[0]
    A = s.shape[1]
    n_mols = acc.shape[0]
    MC = 500
    for h in range(n_mols // MC):
        iota = lax.broadcasted_iota(jnp.int32, (MC, A), 0) + h * MC
        ohT = (jnp.broadcast_to(s, (MC, A)) == iota).astype(jnp.float32)
        acc[pl.ds(h * MC, MC), :] += jnp.dot(
            ohT, ah, preferred_element_type=jnp.float32)
        cnt[pl.ds(h * MC, MC), :] += jnp.sum(ohT, axis=1, keepdims=True)

    @pl.when(i == npg - 1)
    def _():
        out_ref[...] = acc[...] / jnp.maximum(cnt[...], 1.0)


def _tc_readout(f_atoms, am, seg, W_o, b_o, n_mols=2000):
    na = f_atoms.shape[0]
    A = 1000
    seg3 = seg.reshape(na // A, 1, A)
    woa = W_o[:H]
    wom = W_o[H:]
    bor = b_o.reshape(1, H)
    return pl.pallas_call(
        _readout_body,
        grid=(na // A,),
        in_specs=[
            pl.BlockSpec((A, H), lambda i: (i, 0)),
            pl.BlockSpec((A, H), lambda i: (i, 0)),
            pl.BlockSpec((1, 1, A), lambda i: (i, 0, 0)),
            pl.BlockSpec((H, H), lambda i: (0, 0)),
            pl.BlockSpec((H, H), lambda i: (0, 0)),
            pl.BlockSpec((1, H), lambda i: (0, 0)),
        ],
        out_specs=pl.BlockSpec((n_mols, H), lambda i: (0, 0)),
        out_shape=jax.ShapeDtypeStruct((n_mols, H), jnp.float32),
        scratch_shapes=[
            pltpu.VMEM((n_mols, H), jnp.float32),
            pltpu.VMEM((n_mols, 1), jnp.float32),
        ],
        compiler_params=pltpu.CompilerParams(
            dimension_semantics=("arbitrary",)),
    )(f_atoms, am, seg3, woa, wom, bor)


def kernel(f_atoms, f_bonds, a2b, b2a, b2revb, seg, W_i, W_h, W_o, b_o):
    a2b_flat = a2b.reshape(-1)

    inp, hm = _tc_mm0(f_bonds, W_i, W_h)
    ahm0 = _sc_gathersum(hm, a2b_flat)
    g0 = _sc_combine(ahm0, hm, b2a, b2revb)
    hm1 = _tc_mm1(inp, g0, W_h)
    ahm1 = _sc_gathersum(hm1, a2b_flat)
    msg2 = _sc_combine(ahm1, hm1, b2a, b2revb, inp=inp)
    am2 = _sc_gathersum(msg2, a2b_flat)
    return _tc_readout(f_atoms, am2, seg, W_o, b_o)
